# baseline (speedup 1.0000x reference)
import functools

import jax
import jax.numpy as jnp
from jax import lax
from jax.experimental import pallas as pl
from jax.experimental.pallas import tpu as pltpu
from jax.experimental.pallas import tpu_sc as plsc

G = 128
K = 2
NB = 4


def kernel(x, table):
    B0, B1 = x.shape
    V, D = table.shape
    B = B0 * B1

    info = plsc.get_sparse_core_info()
    NC = info.num_cores
    NW = NC * info.num_subcores
    groups_total = B // G
    g_per_w = groups_total // NW
    n_chunks = g_per_w // K
    assert groups_total % NW == 0 and g_per_w % K == 0 and n_chunks % NB == 0

    idx2d = x.reshape(groups_total, G).astype(jnp.int32)

    mesh = plsc.VectorSubcoreMesh(core_axis_name="c", subcore_axis_name="s")

    @functools.partial(
        pl.kernel,
        mesh=mesh,
        out_type=jax.ShapeDtypeStruct((groups_total, G, D), jnp.float32),
        scratch_types=[
            pltpu.VMEM((NB, K, G), jnp.int32),
            pltpu.VMEM((NB, K, G, D), jnp.float32),
            [pltpu.SemaphoreType.DMA] * NB,
            [pltpu.SemaphoreType.DMA] * NB,
        ],
        compiler_params=pltpu.CompilerParams(use_tc_tiling_on_sc=False),
    )
    def emb(idx_hbm, table_hbm, out_hbm, idx_v, rows_v, gsems, osems):
        wid = lax.axis_index("s") * NC + lax.axis_index("c")
        g0 = wid * g_per_w

        def fire(c, b):
            pltpu.sync_copy(idx_hbm.at[pl.ds(g0 + c * K, K)], idx_v.at[b])
            for j in range(K):
                pltpu.async_copy(
                    table_hbm.at[idx_v.at[b, j]], rows_v.at[b, j], gsems[b]
                )

        def wait_gathers(b):
            for j in range(K):
                pltpu.make_async_copy(
                    table_hbm.at[idx_v.at[b, j]], rows_v.at[b, j], gsems[b]
                ).wait()

        def put_out(c, b):
            pltpu.async_copy(
                rows_v.at[b], out_hbm.at[pl.ds(g0 + c * K, K)], osems[b]
            )

        def wait_out(c, b):
            pltpu.make_async_copy(
                rows_v.at[b], out_hbm.at[pl.ds(g0 + c * K, K)], osems[b]
            ).wait()

        for b in range(NB):
            fire(b, b)

        def body(cc, carry):
            for b in range(NB):
                c = cc + b
                wait_gathers(b)
                put_out(c, b)
                wait_out(c, b)
                fire(c + NB, b)
            return carry

        lax.fori_loop(0, n_chunks // NB - 1, lambda i, car: body(i * NB, car), 0)

        for b in range(NB):
            c = n_chunks - NB + b
            wait_gathers(b)
            put_out(c, b)
            wait_out(c, b)

    out = emb(idx2d, table)
    return out.reshape(B0, B1, D)


# speedup vs baseline: 1.3278x; 1.3278x over previous
import functools

import jax
import jax.numpy as jnp
from jax import lax
from jax.experimental import pallas as pl
from jax.experimental.pallas import tpu as pltpu
from jax.experimental.pallas import tpu_sc as plsc

G = 128
K = 2
NB = 4


def kernel(x, table):
    B0, B1 = x.shape
    V, D = table.shape
    B = B0 * B1

    info = plsc.get_sparse_core_info()
    NC = info.num_cores
    NW = NC * info.num_subcores
    groups_total = B // G
    g_per_w = groups_total // NW
    n_chunks = g_per_w // K
    assert groups_total % NW == 0 and g_per_w % K == 0 and n_chunks % NB == 0

    idx2d = x.reshape(groups_total, G).astype(jnp.int32)

    mesh = plsc.VectorSubcoreMesh(core_axis_name="c", subcore_axis_name="s")

    @functools.partial(
        pl.kernel,
        mesh=mesh,
        out_type=jax.ShapeDtypeStruct((groups_total, G, 2 * D), jnp.float32),
        scratch_types=[
            pltpu.VMEM((NB, K, G), jnp.int32),
            pltpu.VMEM((NB, K, G, D), jnp.float32),
            [pltpu.SemaphoreType.DMA] * NB,
            [pltpu.SemaphoreType.DMA] * NB,
        ],
        compiler_params=pltpu.CompilerParams(use_tc_tiling_on_sc=False),
    )
    def emb(idx_hbm, table_hbm, out_hbm, idx_v, rows_v, gsems, osems):
        wid = lax.axis_index("s") * NC + lax.axis_index("c")
        g0 = wid * g_per_w

        def fire(c, b):
            pltpu.sync_copy(idx_hbm.at[pl.ds(g0 + c * K, K)], idx_v.at[b])
            for j in range(K):
                pltpu.async_copy(
                    table_hbm.at[idx_v.at[b, j]], rows_v.at[b, j], gsems[b]
                )

        def wait_gathers(b):
            for j in range(K):
                pltpu.make_async_copy(
                    table_hbm.at[idx_v.at[b, j]], rows_v.at[b, j], gsems[b]
                ).wait()

        def put_out(c, b):
            pltpu.async_copy(
                rows_v.at[b],
                out_hbm.at[pl.ds(g0 + c * K, K), slice(None), pl.ds(0, D)],
                osems[b],
            )

        def wait_out(c, b):
            pltpu.make_async_copy(
                rows_v.at[b],
                out_hbm.at[pl.ds(g0 + c * K, K), slice(None), pl.ds(0, D)],
                osems[b],
            ).wait()

        for b in range(NB):
            fire(b, b)

        def body(cc, carry):
            for b in range(NB):
                c = cc + b
                wait_gathers(b)
                put_out(c, b)
                wait_out(c, b)
                fire(c + NB, b)
            return carry

        lax.fori_loop(0, n_chunks // NB - 1, lambda i, car: body(i * NB, car), 0)

        for b in range(NB):
            c = n_chunks - NB + b
            wait_gathers(b)
            put_out(c, b)
            wait_out(c, b)

    out = emb(idx2d, table)
    return out.reshape(B0, B1, 2 * D)[:, :, :D]
